# baseline (speedup 1.0000x reference)
import jax
import jax.numpy as jnp
from jax.experimental import pallas as pl
from jax.experimental.pallas import tpu as pltpu

N = 4096
H = 24
Z = 64
BM = 256
NB = N // BM
BR = 512
NR = N // BR


def _copy(adj_hbm, buf_ref, sem, blk, slot):
    return pltpu.make_async_copy(
        adj_hbm.at[pl.ds(blk * BM, BM), :], buf_ref.at[slot], sem.at[slot])


def _gcn_body(x0_ref, adj_hbm, w1_ref, b1_ref, w2_ref, b2_ref,
              out_ref, y1_ref, y2_ref, adjb_ref, acc_ref, buf_ref, sem):
    i = pl.program_id(0)

    @pl.when(i == 0)
    def _prologue():
        _copy(adj_hbm, buf_ref, sem, 0, 0).start()
        _copy(adj_hbm, buf_ref, sem, 1, 1).start()
        y1 = jax.lax.dot_general(
            x0_ref[...], w1_ref[...],
            dimension_numbers=(((0,), (0,)), ((), ())),
            preferred_element_type=jnp.float32)
        y1_ref[...] = y1.astype(jnp.bfloat16)

    @pl.when(i + 2 < NB)
    def _prefetch():
        _copy(adj_hbm, buf_ref, sem, i + 2, jax.lax.rem(i + 2, 3)).start()

    slot = jax.lax.rem(i, 3)
    _copy(adj_hbm, buf_ref, sem, i, slot).wait()
    ab = buf_ref[slot].astype(jnp.bfloat16)
    off = pl.multiple_of(i * BM, BM)
    adjb_ref[pl.ds(off, BM), :] = ab
    h1 = jnp.dot(ab, y1_ref[...], preferred_element_type=jnp.float32)
    s1 = jnp.maximum(h1 + b1_ref[...][None, :], 0.0)
    y2_ref[pl.ds(off, BM), :] = jnp.dot(
        s1.astype(jnp.bfloat16), w2_ref[...].astype(jnp.bfloat16),
        preferred_element_type=jnp.float32,
    ).astype(jnp.bfloat16)

    for k in range(NB - 1):
        @pl.when(i == k)
        def _l_piece(k=k):
            P = k * BM
            if k > 0:
                acc_ref[:P, :] += jnp.dot(
                    adjb_ref[:P, P:P + BM], y2_ref[P:P + BM, :],
                    preferred_element_type=jnp.float32)
            acc_ref[P:P + BM, :] = jnp.dot(
                adjb_ref[P:P + BM, :P + BM], y2_ref[:P + BM, :],
                preferred_element_type=jnp.float32)

    @pl.when(i == NB - 1)
    def _final():
        P = (NB - 1) * BM
        HALF = P // 2
        b2v = b2_ref[...][:, None]
        acc_ref[P:, :] = jnp.dot(
            adjb_ref[P:, :], y2_ref[...],
            preferred_element_type=jnp.float32)
        acc_ref[:HALF, :] += jnp.dot(
            adjb_ref[:HALF, P:], y2_ref[P:, :],
            preferred_element_type=jnp.float32)
        for r in range(HALF // BR):
            roff = r * BR
            out_ref[:, pl.ds(roff, BR)] = (
                jnp.transpose(acc_ref[pl.ds(roff, BR), :]) + b2v)
        acc_ref[HALF:P, :] += jnp.dot(
            adjb_ref[HALF:P, P:], y2_ref[P:, :],
            preferred_element_type=jnp.float32)
        for r in range(HALF // BR, NR):
            roff = r * BR
            out_ref[:, pl.ds(roff, BR)] = (
                jnp.transpose(acc_ref[pl.ds(roff, BR), :]) + b2v)


def _gcn(x0, Adj, W1, b1, W2, b2, interpret=False):
    return pl.pallas_call(
        _gcn_body,
        grid=(NB,),
        in_specs=[
            pl.BlockSpec((H, N), lambda i: (0, 0)),
            pl.BlockSpec(memory_space=pltpu.MemorySpace.HBM),
            pl.BlockSpec((H, Z), lambda i: (0, 0)),
            pl.BlockSpec((Z,), lambda i: (0,)),
            pl.BlockSpec((Z, H), lambda i: (0, 0)),
            pl.BlockSpec((H,), lambda i: (0,)),
        ],
        out_specs=pl.BlockSpec((H, N), lambda i: (0, 0)),
        out_shape=jax.ShapeDtypeStruct((H, N), jnp.float32),
        scratch_shapes=[
            pltpu.VMEM((N, Z), jnp.bfloat16),
            pltpu.VMEM((N, H), jnp.bfloat16),
            pltpu.VMEM((N, N), jnp.bfloat16),
            pltpu.VMEM((N, H), jnp.float32),
            pltpu.VMEM((3, BM, N), jnp.float32),
            pltpu.SemaphoreType.DMA((3,)),
        ],
        compiler_params=pltpu.CompilerParams(
            vmem_limit_bytes=64 * 1024 * 1024),
        interpret=interpret,
    )(x0, Adj, W1, b1, W2, b2)


def kernel(X, A_q, A_h, Adj, W1, b1, W2, b2):
    out = _gcn(X[0], Adj, W1, b1, W2, b2)
    return out[None]


# speedup vs baseline: 1.0663x; 1.0663x over previous
import jax
import jax.numpy as jnp
from jax.experimental import pallas as pl
from jax.experimental.pallas import tpu as pltpu

N = 4096
H = 24
Z = 64
BM = 512
NB = N // BM
BR = 512
NR = N // BR


def _copy(adj_hbm, buf_ref, sem, blk, slot):
    return pltpu.make_async_copy(
        adj_hbm.at[pl.ds(blk * BM, BM), :], buf_ref.at[slot], sem.at[slot])


def _gcn_body(x0_ref, adj_hbm, w1_ref, b1_ref, w2_ref, b2_ref,
              out_ref, y1_ref, y2_ref, adjb_ref, acc_ref, buf_ref, sem):
    i = pl.program_id(0)

    @pl.when(i == 0)
    def _prologue():
        _copy(adj_hbm, buf_ref, sem, 0, 0).start()
        _copy(adj_hbm, buf_ref, sem, 1, 1).start()
        y1 = jax.lax.dot_general(
            x0_ref[...], w1_ref[...],
            dimension_numbers=(((0,), (0,)), ((), ())),
            preferred_element_type=jnp.float32)
        y1_ref[...] = y1.astype(jnp.bfloat16)

    @pl.when(i + 2 < NB)
    def _prefetch():
        _copy(adj_hbm, buf_ref, sem, i + 2, jax.lax.rem(i + 2, 3)).start()

    slot = jax.lax.rem(i, 3)
    _copy(adj_hbm, buf_ref, sem, i, slot).wait()
    ab = buf_ref[slot].astype(jnp.bfloat16)
    off = pl.multiple_of(i * BM, BM)
    adjb_ref[pl.ds(off, BM), :] = ab
    h1 = jnp.dot(ab, y1_ref[...], preferred_element_type=jnp.float32)
    s1 = jnp.maximum(h1 + b1_ref[...][None, :], 0.0)
    y2_ref[pl.ds(off, BM), :] = jnp.dot(
        s1.astype(jnp.bfloat16), w2_ref[...].astype(jnp.bfloat16),
        preferred_element_type=jnp.float32,
    ).astype(jnp.bfloat16)

    for k in range(NB - 1):
        @pl.when(i == k)
        def _l_piece(k=k):
            P = k * BM
            if k > 0:
                acc_ref[:P, :] += jnp.dot(
                    adjb_ref[:P, P:P + BM], y2_ref[P:P + BM, :],
                    preferred_element_type=jnp.float32)
            acc_ref[P:P + BM, :] = jnp.dot(
                adjb_ref[P:P + BM, :P + BM], y2_ref[:P + BM, :],
                preferred_element_type=jnp.float32)

    @pl.when(i == NB - 1)
    def _final():
        P = (NB - 1) * BM
        HALF = P // 2
        b2v = b2_ref[...][:, None]
        acc_ref[:HALF, :] += jnp.dot(
            adjb_ref[:HALF, P:], y2_ref[P:, :],
            preferred_element_type=jnp.float32)
        for r in range(HALF // BR):
            roff = r * BR
            out_ref[:, pl.ds(roff, BR)] = (
                jnp.transpose(acc_ref[pl.ds(roff, BR), :]) + b2v)
        acc_ref[HALF:P, :] += jnp.dot(
            adjb_ref[HALF:P, P:], y2_ref[P:, :],
            preferred_element_type=jnp.float32)
        acc_ref[P:, :] = jnp.dot(
            adjb_ref[P:, :], y2_ref[...],
            preferred_element_type=jnp.float32)
        for r in range(HALF // BR, NR):
            roff = r * BR
            out_ref[:, pl.ds(roff, BR)] = (
                jnp.transpose(acc_ref[pl.ds(roff, BR), :]) + b2v)


def _gcn(x0, Adj, W1, b1, W2, b2, interpret=False):
    return pl.pallas_call(
        _gcn_body,
        grid=(NB,),
        in_specs=[
            pl.BlockSpec((H, N), lambda i: (0, 0)),
            pl.BlockSpec(memory_space=pltpu.MemorySpace.HBM),
            pl.BlockSpec((H, Z), lambda i: (0, 0)),
            pl.BlockSpec((Z,), lambda i: (0,)),
            pl.BlockSpec((Z, H), lambda i: (0, 0)),
            pl.BlockSpec((H,), lambda i: (0,)),
        ],
        out_specs=pl.BlockSpec((H, N), lambda i: (0, 0)),
        out_shape=jax.ShapeDtypeStruct((H, N), jnp.float32),
        scratch_shapes=[
            pltpu.VMEM((N, Z), jnp.bfloat16),
            pltpu.VMEM((N, H), jnp.bfloat16),
            pltpu.VMEM((N, N), jnp.bfloat16),
            pltpu.VMEM((N, H), jnp.float32),
            pltpu.VMEM((3, BM, N), jnp.float32),
            pltpu.SemaphoreType.DMA((3,)),
        ],
        compiler_params=pltpu.CompilerParams(
            vmem_limit_bytes=64 * 1024 * 1024),
        interpret=interpret,
    )(x0, Adj, W1, b1, W2, b2)


def kernel(X, A_q, A_h, Adj, W1, b1, W2, b2):
    out = _gcn(X[0], Adj, W1, b1, W2, b2)
    return out[None]
